# baseline (speedup 1.0000x reference)
import functools

import jax
import jax.numpy as jnp
from jax import lax
from jax.experimental import pallas as pl
from jax.experimental.pallas import tpu as pltpu
from jax.experimental.pallas import tpu_sc as plsc

B = 16
T_SPEECH = 4096
T_TXT = 512
L = 16
CHUNKS = T_SPEECH // L
ROWS_PER_CORE = 8


def _body(m_hbm, d_hbm, out_hbm, m_v, d_v, o_v):
    c = lax.axis_index("c")
    s = lax.axis_index("s")
    row = c * ROWS_PER_CORE + s

    @pl.when(s < ROWS_PER_CORE)
    def _():
        pltpu.sync_copy(m_hbm.at[row], m_v.at[pl.ds(0, T_SPEECH)])
        pltpu.sync_copy(d_hbm.at[row], d_v)
        m_v[pl.ds(T_SPEECH, L)] = jnp.full((L,), T_TXT, jnp.int32)

        last = jnp.full((L,), L - 1, jnp.int32)

        @plsc.parallel_loop(0, CHUNKS, unroll=8,
                            carry=jnp.zeros((L,), jnp.float32))
        def _loop(k, carry):
            idx = m_v[pl.ds(k * L, L)]
            idxn = m_v[pl.ds(k * L + 1, L)]
            g = plsc.load_gather(d_v, [jnp.maximum(idx - 1, 0)])
            pos = idx > 0
            mel2dur = jnp.where(pos, g, jnp.float32(1.0))
            delta = jnp.where(idxn > idx, jnp.float32(1.0) - mel2dur,
                              jnp.float32(1.0))
            csum = plsc.cumsum(delta)
            excl = csum - delta + carry
            o_v[pl.ds(k * L, L)] = jnp.where(
                pos, excl / mel2dur, jnp.float32(0.0))
            return carry + csum.at[last].get(mode="promise_in_bounds")
        pltpu.sync_copy(o_v, out_hbm.at[row])


@jax.jit
def _run(mel2ph, dur):
    mesh = plsc.VectorSubcoreMesh(core_axis_name="c", subcore_axis_name="s")
    f = pl.kernel(
        _body,
        out_type=jax.ShapeDtypeStruct((B, T_SPEECH), jnp.float32),
        mesh=mesh,
        compiler_params=pltpu.CompilerParams(needs_layout_passes=False),
        scratch_types=[
            pltpu.VMEM((T_SPEECH + L,), jnp.int32),
            pltpu.VMEM((T_TXT,), jnp.float32),
            pltpu.VMEM((T_SPEECH,), jnp.float32),
        ],
    )
    return f(mel2ph, dur)


def kernel(mel2ph, dur):
    return _run(mel2ph.astype(jnp.int32), dur)


# speedup vs baseline: 1.0072x; 1.0072x over previous
import functools

import jax
import jax.numpy as jnp
from jax import lax
from jax.experimental import pallas as pl
from jax.experimental.pallas import tpu as pltpu
from jax.experimental.pallas import tpu_sc as plsc

B = 16
T_SPEECH = 4096
T_TXT = 512
L = 16
HALF = T_SPEECH // 2
HCHUNKS = HALF // L


def _body(m_hbm, d_hbm, out_hbm, m_v, d_v, o_v):
    half = lax.axis_index("c")
    row = lax.axis_index("s")
    ho = half * HALF

    pltpu.sync_copy(m_hbm.at[row], m_v.at[pl.ds(0, T_SPEECH)])
    pltpu.sync_copy(d_hbm.at[row], d_v)
    m_v[pl.ds(T_SPEECH, L)] = jnp.full((L,), T_TXT, jnp.int32)

    @plsc.parallel_loop(0, HCHUNKS, unroll=8,
                        carry=jnp.zeros((L,), jnp.float32))
    def _pre(k, acc):
        idx = m_v[pl.ds(k * L, L)]
        idxn = m_v[pl.ds(k * L + 1, L)]
        g = plsc.load_gather(d_v, [jnp.maximum(idx - 1, 0)])
        mel2dur = jnp.where(idx > 0, g, jnp.float32(1.0))
        return acc + jnp.where(idxn > idx, mel2dur, jnp.float32(0.0))

    base = (jnp.float32(HALF) - jnp.sum(_pre)) * half.astype(jnp.float32)
    last = jnp.full((L,), L - 1, jnp.int32)

    @plsc.parallel_loop(0, HCHUNKS, unroll=8,
                        carry=jnp.zeros((L,), jnp.float32) + base)
    def _loop(k, carry):
        idx = m_v[pl.ds(ho + k * L, L)]
        idxn = m_v[pl.ds(ho + k * L + 1, L)]
        g = plsc.load_gather(d_v, [jnp.maximum(idx - 1, 0)])
        pos = idx > 0
        mel2dur = jnp.where(pos, g, jnp.float32(1.0))
        delta = jnp.where(idxn > idx, jnp.float32(1.0) - mel2dur,
                          jnp.float32(1.0))
        csum = plsc.cumsum(delta)
        excl = csum - delta + carry
        o_v[pl.ds(k * L, L)] = jnp.where(
            pos, excl / mel2dur, jnp.float32(0.0))
        return carry + csum.at[last].get(mode="promise_in_bounds")

    pltpu.sync_copy(o_v, out_hbm.at[row, pl.ds(ho, HALF)])


@jax.jit
def _run(mel2ph, dur):
    mesh = plsc.VectorSubcoreMesh(core_axis_name="c", subcore_axis_name="s")
    f = pl.kernel(
        _body,
        out_type=jax.ShapeDtypeStruct((B, T_SPEECH), jnp.float32),
        mesh=mesh,
        compiler_params=pltpu.CompilerParams(needs_layout_passes=False),
        scratch_types=[
            pltpu.VMEM((T_SPEECH + L,), jnp.int32),
            pltpu.VMEM((T_TXT,), jnp.float32),
            pltpu.VMEM((HALF,), jnp.float32),
        ],
    )
    return f(mel2ph, dur)


def kernel(mel2ph, dur):
    return _run(mel2ph.astype(jnp.int32), dur)


# speedup vs baseline: 1.0316x; 1.0242x over previous
import functools

import jax
import jax.numpy as jnp
from jax import lax
from jax.experimental import pallas as pl
from jax.experimental.pallas import tpu as pltpu
from jax.experimental.pallas import tpu_sc as plsc

B = 16
T_SPEECH = 4096
T_TXT = 512
L = 16
HALF = T_SPEECH // 2
HCHUNKS = HALF // L


def _body(m_hbm, d_hbm, out_hbm, m_v, d_v, o_v, sem_m, sem_d):
    half = lax.axis_index("c")
    row = lax.axis_index("s")
    ho = half * HALF

    cp_m = pltpu.async_copy(m_hbm.at[row], m_v.at[pl.ds(0, T_SPEECH)], sem_m)
    cp_d = pltpu.async_copy(d_hbm.at[row], d_v, sem_d)
    m_v[pl.ds(T_SPEECH, L)] = jnp.full((L,), T_TXT, jnp.int32)
    cp_m.wait()
    cp_d.wait()

    @plsc.parallel_loop(0, HCHUNKS, unroll=8,
                        carry=jnp.zeros((L,), jnp.float32))
    def _pre(k, acc):
        idx = m_v[pl.ds(k * L, L)]
        idxn = m_v[pl.ds(k * L + 1, L)]
        g = plsc.load_gather(d_v, [jnp.maximum(idx - 1, 0)])
        mel2dur = jnp.where(idx > 0, g, jnp.float32(1.0))
        return acc + jnp.where(idxn > idx, mel2dur, jnp.float32(0.0))

    base = (jnp.float32(HALF) - jnp.sum(_pre)) * half.astype(jnp.float32)
    last = jnp.full((L,), L - 1, jnp.int32)

    @plsc.parallel_loop(0, HCHUNKS, unroll=8,
                        carry=jnp.zeros((L,), jnp.float32) + base)
    def _loop(k, carry):
        idx = m_v[pl.ds(ho + k * L, L)]
        idxn = m_v[pl.ds(ho + k * L + 1, L)]
        g = plsc.load_gather(d_v, [jnp.maximum(idx - 1, 0)])
        pos = idx > 0
        mel2dur = jnp.where(pos, g, jnp.float32(1.0))
        delta = jnp.where(idxn > idx, jnp.float32(1.0) - mel2dur,
                          jnp.float32(1.0))
        csum = plsc.cumsum(delta)
        excl = csum - delta + carry
        o_v[pl.ds(k * L, L)] = jnp.where(
            pos, excl / mel2dur, jnp.float32(0.0))
        return carry + csum.at[last].get(mode="promise_in_bounds")

    pltpu.sync_copy(o_v, out_hbm.at[row, pl.ds(ho, HALF)])


@jax.jit
def _run(mel2ph, dur):
    mesh = plsc.VectorSubcoreMesh(core_axis_name="c", subcore_axis_name="s")
    f = pl.kernel(
        _body,
        out_type=jax.ShapeDtypeStruct((B, T_SPEECH), jnp.float32),
        mesh=mesh,
        compiler_params=pltpu.CompilerParams(needs_layout_passes=False),
        scratch_types=[
            pltpu.VMEM((T_SPEECH + L,), jnp.int32),
            pltpu.VMEM((T_TXT,), jnp.float32),
            pltpu.VMEM((HALF,), jnp.float32),
            pltpu.SemaphoreType.DMA,
            pltpu.SemaphoreType.DMA,
        ],
    )
    return f(mel2ph, dur)


def kernel(mel2ph, dur):
    return _run(mel2ph.astype(jnp.int32), dur)
